# Initial kernel scaffold; baseline (speedup 1.0000x reference)
import functools

import jax
import jax.numpy as jnp
from jax import lax
from jax.experimental import pallas as pl
from jax.experimental.pallas import tpu as pltpu
from jax.experimental.pallas import tpu_sc as plsc

N = 10000
E = 320000
D_IN = 128
H1 = 32
H2 = 16
D_OUT = 2

NC = 2
NS = 16
NW = NC * NS

W = 128
STEPS = 79
EPW = W * STEPS
E_PAD = EPW * NW
N_PAD = 10112
RPT = N_PAD // NS
N_SPARE = N_PAD - N

_f32 = jnp.float32


def _mesh():
    return plsc.VectorSubcoreMesh(core_axis_name="c", subcore_axis_name="s")


@functools.partial(
    pl.kernel,
    out_type=jax.ShapeDtypeStruct((NC, N_PAD), _f32),
    mesh=_mesh(),
    scratch_types=[
        pltpu.VMEM_SHARED((N_PAD,), _f32),
        pltpu.VMEM((W,), jnp.int32),
        pltpu.VMEM((W,), _f32),
        pltpu.VMEM((W,), _f32),
    ],
)
def _hist(dst_hbm, out_hbm, acc_sp, didx, ones, zeros):
    c = lax.axis_index("c")
    s = lax.axis_index("s")
    wid = s * NC + c
    rbase = s * RPT
    for j in range(W // 16):
        ones[pl.ds(j * 16, 16)] = jnp.ones((16,), _f32)
        zeros[pl.ds(j * 16, 16)] = jnp.zeros((16,), _f32)
    for k in range(RPT // W):
        pltpu.sync_copy(zeros, acc_sp.at[pl.ds(rbase + k * W, W)])
    pltpu.sync_copy(zeros.at[pl.ds(0, RPT % W)],
                    acc_sp.at[pl.ds(rbase + (RPT // W) * W, RPT % W)])
    plsc.subcore_barrier()
    ebase = wid * EPW

    def step(t, carry):
        off = pl.multiple_of(ebase + t * W, 8)
        pltpu.sync_copy(dst_hbm.at[pl.ds(off, W)], didx)
        pltpu.sync_copy(ones, acc_sp.at[didx], add=True)
        return carry

    lax.fori_loop(0, STEPS, step, 0)
    plsc.subcore_barrier()
    pltpu.sync_copy(acc_sp.at[pl.ds(rbase, RPT)], out_hbm.at[c, pl.ds(rbase, RPT)])


def _make_agg(D):
    @functools.partial(
        pl.kernel,
        out_type=jax.ShapeDtypeStruct((NC, N_PAD, D), _f32),
        mesh=_mesh(),
        scratch_types=[
            pltpu.VMEM_SHARED((N_PAD, D), _f32),
            pltpu.VMEM_SHARED((N_PAD, D), _f32),
            pltpu.VMEM((W,), jnp.int32),
            pltpu.VMEM((W,), jnp.int32),
            pltpu.VMEM((W, D), _f32),
            pltpu.SemaphoreType.DMA,
        ],
    )
    def agg(xs_hbm, src_hbm, dst_hbm, out_hbm, xs_sp, acc_sp, sidx, didx, rows, sem):
        c = lax.axis_index("c")
        s = lax.axis_index("s")
        wid = s * NC + c
        rbase = s * RPT
        pltpu.sync_copy(xs_hbm.at[pl.ds(rbase, RPT)], xs_sp.at[pl.ds(rbase, RPT)])

        def zrow(i, carry):
            for j in range(D // 16):
                rows[i, pl.ds(j * 16, 16)] = jnp.zeros((16,), _f32)
            return carry

        lax.fori_loop(0, W, zrow, 0)
        for k in range(RPT // W):
            pltpu.sync_copy(rows, acc_sp.at[pl.ds(rbase + k * W, W)])
        pltpu.sync_copy(rows.at[pl.ds(0, RPT % W)],
                        acc_sp.at[pl.ds(rbase + (RPT // W) * W, RPT % W)])
        plsc.subcore_barrier()

        ebase = wid * EPW

        def step(t, carry):
            off = pl.multiple_of(ebase + t * W, 8)
            pltpu.sync_copy(src_hbm.at[pl.ds(off, W)], sidx)
            pltpu.sync_copy(dst_hbm.at[pl.ds(off, W)], didx)
            pltpu.async_copy(xs_sp.at[sidx], rows, sem).wait()
            pltpu.sync_copy(rows, acc_sp.at[didx], add=True)
            return carry

        lax.fori_loop(0, STEPS, step, 0)
        plsc.subcore_barrier()
        pltpu.sync_copy(acc_sp.at[pl.ds(rbase, RPT)],
                        out_hbm.at[c, pl.ds(rbase, RPT)])

    return agg


_agg32 = _make_agg(H1)
_agg16 = _make_agg(H2)


def _prep_body(degp_ref, x_ref, w1_ref, dinv_ref, xs1_ref):
    deg = jnp.sum(degp_ref[...], axis=1, keepdims=True) + 1.0
    dinv = lax.rsqrt(deg)
    mask = (lax.broadcasted_iota(jnp.int32, (N_PAD, 1), 0) < N).astype(_f32)
    dinv = dinv * mask
    dinv_ref[...] = dinv
    xs1_ref[...] = jnp.dot(x_ref[...], w1_ref[...],
                           preferred_element_type=_f32) * dinv


_prep = pl.pallas_call(
    _prep_body,
    out_shape=(jax.ShapeDtypeStruct((N_PAD, 1), _f32),
               jax.ShapeDtypeStruct((N_PAD, H1), _f32)),
)


def _mid1_body(agg_ref, xs_ref, dinv_ref, b_ref, w_ref, out_ref):
    t = agg_ref[0] + agg_ref[1] + xs_ref[...]
    h = jnp.maximum(dinv_ref[...] * t + b_ref[...], 0.0)
    out_ref[...] = jnp.dot(h, w_ref[...], preferred_element_type=_f32) * dinv_ref[...]


_mid1 = pl.pallas_call(
    _mid1_body,
    out_shape=jax.ShapeDtypeStruct((N_PAD, H2), _f32),
)


def _mid2_body(agg_ref, xs_ref, dinv_ref, b_ref, out_ref):
    t = agg_ref[0] + agg_ref[1] + xs_ref[...]
    h = jnp.maximum(dinv_ref[...] * t + b_ref[...], 0.0)
    out_ref[...] = dinv_ref[...] * h


_mid2 = pl.pallas_call(
    _mid2_body,
    out_shape=jax.ShapeDtypeStruct((N_PAD, H2), _f32),
)


def _final_body(agg_ref, xs_ref, dinv_ref, w_ref, b_ref, out_ref):
    t = agg_ref[0] + agg_ref[1] + xs_ref[...]
    z = jnp.dot(dinv_ref[...] * t, w_ref[...],
                preferred_element_type=_f32) + b_ref[...]
    m = jnp.max(z, axis=1, keepdims=True)
    e = jnp.exp(z - m)
    out_ref[...] = (z - m) - jnp.log(jnp.sum(e, axis=1, keepdims=True))


_final = pl.pallas_call(
    _final_body,
    out_shape=jax.ShapeDtypeStruct((N_PAD, D_OUT), _f32),
)


def kernel(x, edge_index, W1, b1, W2, b2, W3, b3):
    pad_ids = jnp.arange(E_PAD - E, dtype=jnp.int32)
    src_p = jnp.concatenate([edge_index[0], N + pad_ids % N_SPARE])
    dst_p = jnp.concatenate([edge_index[1], N + pad_ids % N_SPARE])
    x_p = jnp.pad(x, ((0, N_PAD - N), (0, 0)))

    degp = _hist(dst_p)
    dinv, xs1 = _prep(degp.T, x_p, W1)
    agg1 = _agg32(xs1, src_p, dst_p)
    xs2 = _mid1(agg1, xs1, dinv, b1.reshape(1, H1), W2)
    agg2 = _agg16(xs2, src_p, dst_p)
    xs3 = _mid2(agg2, xs2, dinv, b2.reshape(1, H2))
    agg3 = _agg16(xs3, src_p, dst_p)
    out = _final(agg3, xs3, dinv, W3, b3.reshape(1, D_OUT))
    return out[:N]


# speedup vs baseline: 23.4718x; 23.4718x over previous
import functools

import jax
import jax.numpy as jnp
from jax import lax
from jax.experimental import pallas as pl
from jax.experimental.pallas import tpu as pltpu
from jax.experimental.pallas import tpu_sc as plsc

N = 10000
E = 320000
D_IN = 128
H1 = 32
H2 = 16
D_OUT = 2

NC = 2
NS = 16
NW = NC * NS

W = 128
STEPS = 79
EPW = W * STEPS
E_PAD = EPW * NW
N_PAD = 10112
RPT = N_PAD // NS
N_SPARE = N_PAD - N

_f32 = jnp.float32


def _mesh():
    return plsc.VectorSubcoreMesh(core_axis_name="c", subcore_axis_name="s")


def _hist_body(dst_hbm, out_hbm, acc_sp, didx, ones, zeros, vbuf):
    c = lax.axis_index("c")
    s = lax.axis_index("s")
    wid = s * NC + c
    rbase = s * RPT
    for j in range(W // 16):
        ones[pl.ds(j * 16, 16)] = jnp.ones((16,), _f32)
        zeros[pl.ds(j * 16, 16)] = jnp.zeros((16,), _f32)
    for k in range(RPT // W):
        pltpu.sync_copy(zeros, acc_sp.at[pl.ds(rbase + k * W, W)])
    pltpu.sync_copy(zeros.at[pl.ds(0, RPT % W)],
                    acc_sp.at[pl.ds(rbase + (RPT // W) * W, RPT % W)])
    plsc.subcore_barrier()
    ebase = wid * EPW

    def step(t, carry):
        off = pl.multiple_of(ebase + t * W, 8)
        pltpu.sync_copy(dst_hbm.at[pl.ds(off, W)], didx)
        pltpu.sync_copy(ones, acc_sp.at[didx], add=True)
        return carry

    lax.fori_loop(0, STEPS, step, 0)
    plsc.subcore_barrier()
    obase = pl.multiple_of(c * N_PAD + rbase, 8)
    for k in range(RPT // W + 1):
        ln = W if k < RPT // W else RPT % W
        pltpu.sync_copy(acc_sp.at[pl.ds(rbase + k * W, ln)], vbuf.at[pl.ds(0, ln)])
        pltpu.sync_copy(vbuf.at[pl.ds(0, ln)], out_hbm.at[pl.ds(obase + k * W, ln)])


_hist = functools.partial(
    pl.kernel,
    out_type=jax.ShapeDtypeStruct((NC * N_PAD,), _f32),
    mesh=_mesh(),
    scratch_types=[
        pltpu.VMEM_SHARED((N_PAD,), _f32),
        pltpu.VMEM((W,), jnp.int32),
        pltpu.VMEM((W,), _f32),
        pltpu.VMEM((W,), _f32),
        pltpu.VMEM((W,), _f32),
    ],
)(_hist_body)


def _agg_body_for(D):
    def agg(xs_hbm, src_hbm, dst_hbm, out_hbm, xs_sp, acc_sp, sidx, didx, rows, sem):
        c = lax.axis_index("c")
        s = lax.axis_index("s")
        wid = s * NC + c
        rbase = s * RPT
        pltpu.sync_copy(xs_hbm.at[pl.ds(rbase, RPT)], xs_sp.at[pl.ds(rbase, RPT)])

        def zrow(i, carry):
            for j in range(D // 16):
                rows[i, pl.ds(j * 16, 16)] = jnp.zeros((16,), _f32)
            return carry

        lax.fori_loop(0, W, zrow, 0)
        for k in range(RPT // W):
            pltpu.sync_copy(rows, acc_sp.at[pl.ds(rbase + k * W, W)])
        pltpu.sync_copy(rows.at[pl.ds(0, RPT % W)],
                        acc_sp.at[pl.ds(rbase + (RPT // W) * W, RPT % W)])
        plsc.subcore_barrier()

        ebase = wid * EPW

        def step(t, carry):
            off = pl.multiple_of(ebase + t * W, 8)
            pltpu.sync_copy(src_hbm.at[pl.ds(off, W)], sidx)
            pltpu.sync_copy(dst_hbm.at[pl.ds(off, W)], didx)
            pltpu.async_copy(xs_sp.at[sidx], rows, sem).wait()
            pltpu.sync_copy(rows, acc_sp.at[didx], add=True)
            return carry

        lax.fori_loop(0, STEPS, step, 0)
        plsc.subcore_barrier()
        pltpu.sync_copy(acc_sp.at[pl.ds(rbase, RPT)],
                        out_hbm.at[c, pl.ds(rbase, RPT)])

    return agg


def _make_agg(D):
    return functools.partial(
        pl.kernel,
        out_type=jax.ShapeDtypeStruct((NC, N_PAD, D), _f32),
        mesh=_mesh(),
        compiler_params=pltpu.CompilerParams(use_tc_tiling_on_sc=False),
        scratch_types=[
            pltpu.VMEM_SHARED((N_PAD, D), _f32),
            pltpu.VMEM_SHARED((N_PAD, D), _f32),
            pltpu.VMEM((W,), jnp.int32),
            pltpu.VMEM((W,), jnp.int32),
            pltpu.VMEM((W, D), _f32),
            pltpu.SemaphoreType.DMA,
        ],
    )(_agg_body_for(D))


_agg32 = _make_agg(H1)
_agg16 = _make_agg(H2)


def _prep_body(degp_ref, x_ref, w1_ref, dinv_ref, xs1_ref):
    deg = jnp.sum(degp_ref[...], axis=1, keepdims=True) + 1.0
    dinv = lax.rsqrt(deg)
    mask = (lax.broadcasted_iota(jnp.int32, (N_PAD, 1), 0) < N).astype(_f32)
    dinv = dinv * mask
    dinv_ref[...] = dinv
    xs1_ref[...] = jnp.dot(x_ref[...], w1_ref[...],
                           preferred_element_type=_f32) * dinv


_prep = pl.pallas_call(
    _prep_body,
    out_shape=(jax.ShapeDtypeStruct((N_PAD, 1), _f32),
               jax.ShapeDtypeStruct((N_PAD, H1), _f32)),
)


def _mid1_body(agg_ref, xs_ref, dinv_ref, b_ref, w_ref, out_ref):
    t = agg_ref[0] + agg_ref[1] + xs_ref[...]
    h = jnp.maximum(dinv_ref[...] * t + b_ref[...], 0.0)
    out_ref[...] = jnp.dot(h, w_ref[...], preferred_element_type=_f32) * dinv_ref[...]


_mid1 = pl.pallas_call(
    _mid1_body,
    out_shape=jax.ShapeDtypeStruct((N_PAD, H2), _f32),
)


def _mid2_body(agg_ref, xs_ref, dinv_ref, b_ref, out_ref):
    t = agg_ref[0] + agg_ref[1] + xs_ref[...]
    h = jnp.maximum(dinv_ref[...] * t + b_ref[...], 0.0)
    out_ref[...] = dinv_ref[...] * h


_mid2 = pl.pallas_call(
    _mid2_body,
    out_shape=jax.ShapeDtypeStruct((N_PAD, H2), _f32),
)


def _final_body(agg_ref, xs_ref, dinv_ref, w_ref, b_ref, out_ref):
    t = agg_ref[0] + agg_ref[1] + xs_ref[...]
    z = jnp.dot(dinv_ref[...] * t, w_ref[...],
                preferred_element_type=_f32) + b_ref[...]
    m = jnp.max(z, axis=1, keepdims=True)
    e = jnp.exp(z - m)
    out_ref[...] = (z - m) - jnp.log(jnp.sum(e, axis=1, keepdims=True))


_final = pl.pallas_call(
    _final_body,
    out_shape=jax.ShapeDtypeStruct((N_PAD, D_OUT), _f32),
)


def kernel(x, edge_index, W1, b1, W2, b2, W3, b3):
    pad_ids = jnp.arange(E_PAD - E, dtype=jnp.int32)
    src_p = jnp.concatenate([edge_index[0], N + pad_ids % N_SPARE])
    dst_p = jnp.concatenate([edge_index[1], N + pad_ids % N_SPARE])
    x_p = jnp.pad(x, ((0, N_PAD - N), (0, 0)))

    degp = _hist(dst_p).reshape(NC, N_PAD)
    dinv, xs1 = _prep(degp.T, x_p, W1)
    agg1 = _agg32(xs1, src_p, dst_p)
    xs2 = _mid1(agg1, xs1, dinv, b1.reshape(1, H1), W2)
    agg2 = _agg16(xs2, src_p, dst_p)
    xs3 = _mid2(agg2, xs2, dinv, b2.reshape(1, H2))
    agg3 = _agg16(xs3, src_p, dst_p)
    out = _final(agg3, xs3, dinv, W3, b3.reshape(1, D_OUT))
    return out[:N]


# speedup vs baseline: 56.0309x; 2.3872x over previous
import functools

import jax
import jax.numpy as jnp
from jax import lax
from jax.experimental import pallas as pl
from jax.experimental.pallas import tpu as pltpu
from jax.experimental.pallas import tpu_sc as plsc

N = 10000
E = 320000
D_IN = 128
H1 = 32
H2 = 16
D_OUT = 2

NC = 2
NS = 16
NW = NC * NS

W = 128
STEPS = 80
NBUF = 4
GROUPS = STEPS // NBUF
EPW = W * STEPS
E_PAD = EPW * NW
N_PAD = 10112
RPT = N_PAD // NS
N_SPARE = N_PAD - N

_f32 = jnp.float32

_SC_PARAMS = pltpu.CompilerParams(use_tc_tiling_on_sc=False)


def _mesh():
    return plsc.VectorSubcoreMesh(core_axis_name="c", subcore_axis_name="s")


def _hist_body(dst_hbm, out_hbm, acc_sp, didx, ones, zeros, vbuf, sem0, sem1,
               sem2, sem3):
    c = lax.axis_index("c")
    s = lax.axis_index("s")
    wid = s * NC + c
    rbase = s * RPT
    ssem = [sem0, sem1, sem2, sem3]
    pltpu.sync_copy(dst_hbm.at[pl.ds(wid * STEPS, STEPS)], didx)
    for j in range(W // 16):
        ones[pl.ds(j * 16, 16)] = jnp.ones((16,), _f32)
        zeros[pl.ds(j * 16, 16)] = jnp.zeros((16,), _f32)
    for k in range(RPT // W):
        pltpu.sync_copy(zeros, acc_sp.at[pl.ds(rbase + k * W, W)])
    pltpu.sync_copy(zeros.at[pl.ds(0, RPT % W)],
                    acc_sp.at[pl.ds(rbase + (RPT // W) * W, RPT % W)])
    plsc.subcore_barrier()

    def group(g, carry):
        for b in range(NBUF):
            t = g * NBUF + b

            @pl.when(g > 0)
            def _():
                pltpu.make_async_copy(ones, acc_sp.at[didx.at[t]],
                                      ssem[b]).wait()

            pltpu.async_copy(ones, acc_sp.at[didx.at[t]], ssem[b], add=True)
        return carry

    lax.fori_loop(0, GROUPS, group, 0)
    for b in range(NBUF):
        pltpu.make_async_copy(ones, acc_sp.at[didx.at[0]], ssem[b]).wait()
    plsc.subcore_barrier()
    obase = pl.multiple_of(c * N_PAD + rbase, 8)
    for k in range(RPT // W + 1):
        ln = W if k < RPT // W else RPT % W
        pltpu.sync_copy(acc_sp.at[pl.ds(rbase + k * W, ln)], vbuf.at[pl.ds(0, ln)])
        pltpu.sync_copy(vbuf.at[pl.ds(0, ln)], out_hbm.at[pl.ds(obase + k * W, ln)])


_hist = functools.partial(
    pl.kernel,
    out_type=jax.ShapeDtypeStruct((NC * N_PAD,), _f32),
    mesh=_mesh(),
    compiler_params=_SC_PARAMS,
    scratch_types=[
        pltpu.VMEM_SHARED((N_PAD,), _f32),
        pltpu.VMEM((STEPS, W), jnp.int32),
        pltpu.VMEM((W,), _f32),
        pltpu.VMEM((W,), _f32),
        pltpu.VMEM((W,), _f32),
        pltpu.SemaphoreType.DMA,
        pltpu.SemaphoreType.DMA,
        pltpu.SemaphoreType.DMA,
        pltpu.SemaphoreType.DMA,
    ],
)(_hist_body)


def _agg_body_for(D):
    def agg(xs_hbm, src_hbm, dst_hbm, out_hbm, xs_sp, acc_sp, sidx, didx,
            rows0, rows1, rows2, rows3, g0, g1, g2, g3, s0, s1, s2, s3):
        c = lax.axis_index("c")
        s = lax.axis_index("s")
        wid = s * NC + c
        rbase = s * RPT
        rows = [rows0, rows1, rows2, rows3]
        gsem = [g0, g1, g2, g3]
        ssem = [s0, s1, s2, s3]
        pltpu.sync_copy(xs_hbm.at[pl.ds(rbase, RPT)], xs_sp.at[pl.ds(rbase, RPT)])
        pltpu.sync_copy(src_hbm.at[pl.ds(wid * STEPS, STEPS)], sidx)
        pltpu.sync_copy(dst_hbm.at[pl.ds(wid * STEPS, STEPS)], didx)

        def zrow(i, carry):
            for j in range(D // 16):
                rows0[i, pl.ds(j * 16, 16)] = jnp.zeros((16,), _f32)
            return carry

        lax.fori_loop(0, W, zrow, 0)
        for k in range(RPT // W):
            pltpu.sync_copy(rows0, acc_sp.at[pl.ds(rbase + k * W, W)])
        pltpu.sync_copy(rows0.at[pl.ds(0, RPT % W)],
                        acc_sp.at[pl.ds(rbase + (RPT // W) * W, RPT % W)])
        plsc.subcore_barrier()

        pltpu.async_copy(xs_sp.at[sidx.at[0]], rows[0], gsem[0])
        pltpu.async_copy(xs_sp.at[sidx.at[1]], rows[1], gsem[1])

        def group(g, carry):
            for b in range(NBUF):
                t = g * NBUF + b
                b2 = (b + 2) % NBUF
                pltpu.make_async_copy(xs_sp.at[sidx.at[t]], rows[b],
                                      gsem[b]).wait()
                pltpu.async_copy(rows[b], acc_sp.at[didx.at[t]], ssem[b],
                                 add=True)
                if b < 2:
                    @pl.when(g > 0)
                    def _():
                        pltpu.make_async_copy(rows[b2], acc_sp.at[didx.at[t]],
                                              ssem[b2]).wait()

                    pltpu.async_copy(xs_sp.at[sidx.at[t + 2]], rows[b2],
                                     gsem[b2])
                else:
                    @pl.when(g < GROUPS - 1)
                    def _():
                        pltpu.make_async_copy(rows[b2], acc_sp.at[didx.at[t]],
                                              ssem[b2]).wait()
                        pltpu.async_copy(xs_sp.at[sidx.at[t + 2]], rows[b2],
                                         gsem[b2])
            return carry

        lax.fori_loop(0, GROUPS, group, 0)
        for b in range(NBUF):
            pltpu.make_async_copy(rows[b], acc_sp.at[didx.at[0]],
                                  ssem[b]).wait()
        plsc.subcore_barrier()
        pltpu.sync_copy(acc_sp.at[pl.ds(rbase, RPT)],
                        out_hbm.at[c, pl.ds(rbase, RPT)])

    return agg


def _make_agg(D):
    return functools.partial(
        pl.kernel,
        out_type=jax.ShapeDtypeStruct((NC, N_PAD, D), _f32),
        mesh=_mesh(),
        compiler_params=_SC_PARAMS,
        scratch_types=[
            pltpu.VMEM_SHARED((N_PAD, D), _f32),
            pltpu.VMEM_SHARED((N_PAD, D), _f32),
            pltpu.VMEM((STEPS, W), jnp.int32),
            pltpu.VMEM((STEPS, W), jnp.int32),
            pltpu.VMEM((W, D), _f32),
            pltpu.VMEM((W, D), _f32),
            pltpu.VMEM((W, D), _f32),
            pltpu.VMEM((W, D), _f32),
            pltpu.SemaphoreType.DMA,
            pltpu.SemaphoreType.DMA,
            pltpu.SemaphoreType.DMA,
            pltpu.SemaphoreType.DMA,
            pltpu.SemaphoreType.DMA,
            pltpu.SemaphoreType.DMA,
            pltpu.SemaphoreType.DMA,
            pltpu.SemaphoreType.DMA,
        ],
    )(_agg_body_for(D))


_agg32 = _make_agg(H1)
_agg16 = _make_agg(H2)


def _prep_body(degp_ref, x_ref, w1_ref, dinv_ref, xs1_ref):
    deg = jnp.sum(degp_ref[...], axis=1, keepdims=True) + 1.0
    dinv = lax.rsqrt(deg)
    mask = (lax.broadcasted_iota(jnp.int32, (N_PAD, 1), 0) < N).astype(_f32)
    dinv = dinv * mask
    dinv_ref[...] = dinv
    xs1_ref[...] = jnp.dot(x_ref[...], w1_ref[...],
                           preferred_element_type=_f32) * dinv


_prep = pl.pallas_call(
    _prep_body,
    out_shape=(jax.ShapeDtypeStruct((N_PAD, 1), _f32),
               jax.ShapeDtypeStruct((N_PAD, H1), _f32)),
)


def _mid1_body(agg_ref, xs_ref, dinv_ref, b_ref, w_ref, out_ref):
    t = agg_ref[0] + agg_ref[1] + xs_ref[...]
    h = jnp.maximum(dinv_ref[...] * t + b_ref[...], 0.0)
    out_ref[...] = jnp.dot(h, w_ref[...], preferred_element_type=_f32) * dinv_ref[...]


_mid1 = pl.pallas_call(
    _mid1_body,
    out_shape=jax.ShapeDtypeStruct((N_PAD, H2), _f32),
)


def _mid2_body(agg_ref, xs_ref, dinv_ref, b_ref, out_ref):
    t = agg_ref[0] + agg_ref[1] + xs_ref[...]
    h = jnp.maximum(dinv_ref[...] * t + b_ref[...], 0.0)
    out_ref[...] = dinv_ref[...] * h


_mid2 = pl.pallas_call(
    _mid2_body,
    out_shape=jax.ShapeDtypeStruct((N_PAD, H2), _f32),
)


def _final_body(agg_ref, xs_ref, dinv_ref, w_ref, b_ref, out_ref):
    t = agg_ref[0] + agg_ref[1] + xs_ref[...]
    z = jnp.dot(dinv_ref[...] * t, w_ref[...],
                preferred_element_type=_f32) + b_ref[...]
    m = jnp.max(z, axis=1, keepdims=True)
    e = jnp.exp(z - m)
    out_ref[...] = (z - m) - jnp.log(jnp.sum(e, axis=1, keepdims=True))


_final = pl.pallas_call(
    _final_body,
    out_shape=jax.ShapeDtypeStruct((N_PAD, D_OUT), _f32),
)


def kernel(x, edge_index, W1, b1, W2, b2, W3, b3):
    pad_ids = jnp.arange(E_PAD - E, dtype=jnp.int32)
    src_p = jnp.concatenate([edge_index[0], N + pad_ids % N_SPARE])
    dst_p = jnp.concatenate([edge_index[1], N + pad_ids % N_SPARE])
    src_p = src_p.reshape(NW * STEPS, W)
    dst_p = dst_p.reshape(NW * STEPS, W)
    x_p = jnp.pad(x, ((0, N_PAD - N), (0, 0)))

    degp = _hist(dst_p).reshape(NC, N_PAD)
    dinv, xs1 = _prep(degp.T, x_p, W1)
    agg1 = _agg32(xs1, src_p, dst_p)
    xs2 = _mid1(agg1, xs1, dinv, b1.reshape(1, H1), W2)
    agg2 = _agg16(xs2, src_p, dst_p)
    xs3 = _mid2(agg2, xs2, dinv, b2.reshape(1, H2))
    agg3 = _agg16(xs3, src_p, dst_p)
    out = _final(agg3, xs3, dinv, W3, b3.reshape(1, D_OUT))
    return out[:N]


# speedup vs baseline: 59.2626x; 1.0577x over previous
import functools

import jax
import jax.numpy as jnp
from jax import lax
from jax.experimental import pallas as pl
from jax.experimental.pallas import tpu as pltpu
from jax.experimental.pallas import tpu_sc as plsc

N = 10000
E = 320000
D_IN = 128
H1 = 32
H2 = 16
D_OUT = 2

NC = 2
NS = 16
NW = NC * NS

W = 128
STEPS = 80
NBUF = 4
GROUPS = STEPS // NBUF
EPW = W * STEPS
E_PAD = EPW * NW
N_PAD = 10112
RPT = N_PAD // NS
N_SPARE = N_PAD - N

_f32 = jnp.float32

_SC_PARAMS = pltpu.CompilerParams(use_tc_tiling_on_sc=False)


def _mesh():
    return plsc.VectorSubcoreMesh(core_axis_name="c", subcore_axis_name="s")


def _hist_body(dst_hbm, out_hbm, acc_sp, didx, ones, zeros, vbuf, sem0, sem1,
               sem2, sem3):
    c = lax.axis_index("c")
    s = lax.axis_index("s")
    wid = s * NC + c
    rbase = s * RPT
    ssem = [sem0, sem1, sem2, sem3]
    pltpu.sync_copy(dst_hbm.at[pl.ds(wid * STEPS, STEPS)], didx)
    for j in range(W // 16):
        ones[pl.ds(j * 16, 16)] = jnp.ones((16,), _f32)
        zeros[pl.ds(j * 16, 16)] = jnp.zeros((16,), _f32)
    for k in range(RPT // W):
        pltpu.sync_copy(zeros, acc_sp.at[pl.ds(rbase + k * W, W)])
    pltpu.sync_copy(zeros.at[pl.ds(0, RPT % W)],
                    acc_sp.at[pl.ds(rbase + (RPT // W) * W, RPT % W)])
    plsc.subcore_barrier()

    def group(g, carry):
        for b in range(NBUF):
            t = g * NBUF + b

            @pl.when(g > 0)
            def _():
                pltpu.make_async_copy(ones, acc_sp.at[didx.at[t]],
                                      ssem[b]).wait()

            pltpu.async_copy(ones, acc_sp.at[didx.at[t]], ssem[b], add=True)
        return carry

    lax.fori_loop(0, GROUPS, group, 0)
    for b in range(NBUF):
        pltpu.make_async_copy(ones, acc_sp.at[didx.at[0]], ssem[b]).wait()
    plsc.subcore_barrier()
    obase = pl.multiple_of(c * N_PAD + rbase, 8)
    for k in range(RPT // W + 1):
        ln = W if k < RPT // W else RPT % W
        pltpu.sync_copy(acc_sp.at[pl.ds(rbase + k * W, ln)], vbuf.at[pl.ds(0, ln)])
        pltpu.sync_copy(vbuf.at[pl.ds(0, ln)], out_hbm.at[pl.ds(obase + k * W, ln)])


_hist = functools.partial(
    pl.kernel,
    out_type=jax.ShapeDtypeStruct((NC * N_PAD,), _f32),
    mesh=_mesh(),
    compiler_params=_SC_PARAMS,
    scratch_types=[
        pltpu.VMEM_SHARED((N_PAD,), _f32),
        pltpu.VMEM((STEPS, W), jnp.int32),
        pltpu.VMEM((W,), _f32),
        pltpu.VMEM((W,), _f32),
        pltpu.VMEM((W,), _f32),
        pltpu.SemaphoreType.DMA,
        pltpu.SemaphoreType.DMA,
        pltpu.SemaphoreType.DMA,
        pltpu.SemaphoreType.DMA,
    ],
)(_hist_body)


def _agg_body_for(D):
    def agg(xs_hbm, src_hbm, dst_hbm, out_hbm, xs_sp, acc_sp, sidx, didx,
            rows0, rows1, rows2, rows3, g0, g1, g2, g3, s0, s1, s2, s3):
        c = lax.axis_index("c")
        s = lax.axis_index("s")
        wid = s * NC + c
        rbase = s * RPT
        rows = [rows0, rows1, rows2, rows3]
        gsem = [g0, g1, g2, g3]
        ssem = [s0, s1, s2, s3]
        pltpu.sync_copy(xs_hbm.at[pl.ds(rbase, RPT)], xs_sp.at[pl.ds(rbase, RPT)])
        pltpu.sync_copy(src_hbm.at[pl.ds(wid * STEPS, STEPS)], sidx)
        pltpu.sync_copy(dst_hbm.at[pl.ds(wid * STEPS, STEPS)], didx)

        def zrow(i, carry):
            for j in range(D // 16):
                rows0[i, pl.ds(j * 16, 16)] = jnp.zeros((16,), _f32)
            return carry

        lax.fori_loop(0, W, zrow, 0)
        for k in range(RPT // W):
            pltpu.sync_copy(rows0, acc_sp.at[pl.ds(rbase + k * W, W)])
        pltpu.sync_copy(rows0.at[pl.ds(0, RPT % W)],
                        acc_sp.at[pl.ds(rbase + (RPT // W) * W, RPT % W)])
        plsc.subcore_barrier()

        pltpu.async_copy(xs_sp.at[sidx.at[0]], rows[0], gsem[0])
        pltpu.async_copy(xs_sp.at[sidx.at[1]], rows[1], gsem[1])

        def group(g, carry):
            for b in range(NBUF):
                t = g * NBUF + b
                b2 = (b + 2) % NBUF
                pltpu.make_async_copy(xs_sp.at[sidx.at[t]], rows[b],
                                      gsem[b]).wait()
                pltpu.async_copy(rows[b], acc_sp.at[didx.at[t]], ssem[b],
                                 add=True)
                if b < 2:
                    @pl.when(g > 0)
                    def _():
                        pltpu.make_async_copy(rows[b2], acc_sp.at[didx.at[t]],
                                              ssem[b2]).wait()

                    pltpu.async_copy(xs_sp.at[sidx.at[t + 2]], rows[b2],
                                     gsem[b2])
                else:
                    @pl.when(g < GROUPS - 1)
                    def _():
                        pltpu.make_async_copy(rows[b2], acc_sp.at[didx.at[t]],
                                              ssem[b2]).wait()
                        pltpu.async_copy(xs_sp.at[sidx.at[t + 2]], rows[b2],
                                         gsem[b2])
            return carry

        lax.fori_loop(0, GROUPS, group, 0)
        for b in range(NBUF):
            pltpu.make_async_copy(rows[b], acc_sp.at[didx.at[0]],
                                  ssem[b]).wait()
        plsc.subcore_barrier()
        pltpu.sync_copy(acc_sp.at[pl.ds(rbase, RPT)],
                        out_hbm.at[c, pl.ds(rbase, RPT)])

    return agg


def _make_agg(D):
    return functools.partial(
        pl.kernel,
        out_type=jax.ShapeDtypeStruct((NC, N_PAD, D), _f32),
        mesh=_mesh(),
        compiler_params=_SC_PARAMS,
        scratch_types=[
            pltpu.VMEM_SHARED((N_PAD, D), _f32),
            pltpu.VMEM_SHARED((N_PAD, D), _f32),
            pltpu.VMEM((STEPS, W), jnp.int32),
            pltpu.VMEM((STEPS, W), jnp.int32),
            pltpu.VMEM((W, D), _f32),
            pltpu.VMEM((W, D), _f32),
            pltpu.VMEM((W, D), _f32),
            pltpu.VMEM((W, D), _f32),
            pltpu.SemaphoreType.DMA,
            pltpu.SemaphoreType.DMA,
            pltpu.SemaphoreType.DMA,
            pltpu.SemaphoreType.DMA,
            pltpu.SemaphoreType.DMA,
            pltpu.SemaphoreType.DMA,
            pltpu.SemaphoreType.DMA,
            pltpu.SemaphoreType.DMA,
        ],
    )(_agg_body_for(D))


_agg32 = _make_agg(H1)
_agg16 = _make_agg(H2)


def _edges_body(ei_ref, src_ref, dst_ref):
    pad = N + lax.rem(
        lax.broadcasted_iota(jnp.int32, (E_PAD - E,), 0), N_SPARE)
    src_ref[pl.ds(0, E)] = ei_ref[0]
    dst_ref[pl.ds(0, E)] = ei_ref[1]
    src_ref[pl.ds(E, E_PAD - E)] = pad
    dst_ref[pl.ds(E, E_PAD - E)] = pad


_edges = pl.pallas_call(
    _edges_body,
    out_shape=(jax.ShapeDtypeStruct((E_PAD,), jnp.int32),
               jax.ShapeDtypeStruct((E_PAD,), jnp.int32)),
)


def _prep_body(degp_ref, x_ref, w1_ref, dinv_ref, xs1_ref):
    deg = jnp.sum(degp_ref[...], axis=1, keepdims=True) + 1.0
    dinv = lax.rsqrt(deg)
    mask = (lax.broadcasted_iota(jnp.int32, (N_PAD, 1), 0) < N).astype(_f32)
    dinv = dinv * mask
    dinv_ref[...] = dinv
    xs1_ref[...] = jnp.dot(x_ref[...], w1_ref[...],
                           preferred_element_type=_f32) * dinv


_prep = pl.pallas_call(
    _prep_body,
    out_shape=(jax.ShapeDtypeStruct((N_PAD, 1), _f32),
               jax.ShapeDtypeStruct((N_PAD, H1), _f32)),
)


def _mid1_body(agg_ref, xs_ref, dinv_ref, b_ref, w_ref, out_ref):
    t = agg_ref[0] + agg_ref[1] + xs_ref[...]
    h = jnp.maximum(dinv_ref[...] * t + b_ref[...], 0.0)
    out_ref[...] = jnp.dot(h, w_ref[...], preferred_element_type=_f32) * dinv_ref[...]


_mid1 = pl.pallas_call(
    _mid1_body,
    out_shape=jax.ShapeDtypeStruct((N_PAD, H2), _f32),
)


def _mid2_body(agg_ref, xs_ref, dinv_ref, b_ref, out_ref):
    t = agg_ref[0] + agg_ref[1] + xs_ref[...]
    h = jnp.maximum(dinv_ref[...] * t + b_ref[...], 0.0)
    out_ref[...] = dinv_ref[...] * h


_mid2 = pl.pallas_call(
    _mid2_body,
    out_shape=jax.ShapeDtypeStruct((N_PAD, H2), _f32),
)


def _final_body(agg_ref, xs_ref, dinv_ref, w_ref, b_ref, out_ref):
    t = agg_ref[0] + agg_ref[1] + xs_ref[...]
    z = jnp.dot(dinv_ref[...] * t, w_ref[...],
                preferred_element_type=_f32) + b_ref[...]
    m = jnp.max(z, axis=1, keepdims=True)
    e = jnp.exp(z - m)
    out_ref[...] = (z - m) - jnp.log(jnp.sum(e, axis=1, keepdims=True))


_final = pl.pallas_call(
    _final_body,
    out_shape=jax.ShapeDtypeStruct((N_PAD, D_OUT), _f32),
)


def kernel(x, edge_index, W1, b1, W2, b2, W3, b3):
    src_p, dst_p = _edges(edge_index)
    src_p = src_p.reshape(NW * STEPS, W)
    dst_p = dst_p.reshape(NW * STEPS, W)
    x_p = jnp.pad(x, ((0, N_PAD - N), (0, 0)))

    degp = _hist(dst_p).reshape(NC, N_PAD)
    dinv, xs1 = _prep(degp.T, x_p, W1)
    agg1 = _agg32(xs1, src_p, dst_p)
    xs2 = _mid1(agg1, xs1, dinv, b1.reshape(1, H1), W2)
    agg2 = _agg16(xs2, src_p, dst_p)
    xs3 = _mid2(agg2, xs2, dinv, b2.reshape(1, H2))
    agg3 = _agg16(xs3, src_p, dst_p)
    out = _final(agg3, xs3, dinv, W3, b3.reshape(1, D_OUT))
    return out[:N]


# speedup vs baseline: 62.7222x; 1.0584x over previous
import functools

import jax
import jax.numpy as jnp
from jax import lax
from jax.experimental import pallas as pl
from jax.experimental.pallas import tpu as pltpu
from jax.experimental.pallas import tpu_sc as plsc

N = 10000
E = 320000
D_IN = 128
H1 = 32
H2 = 16
D_OUT = 2

NC = 2
NS = 16
NW = NC * NS

W = 128
STEPS = 80
NBUF = 4
GROUPS = STEPS // NBUF
EPW = W * STEPS
E_PAD = EPW * NW
N_PAD = 10112
RPT = N_PAD // NS
N_SPARE = N_PAD - N

_f32 = jnp.float32

_SC_PARAMS = pltpu.CompilerParams(use_tc_tiling_on_sc=False)


def _mesh():
    return plsc.VectorSubcoreMesh(core_axis_name="c", subcore_axis_name="s")


def _hist_body(dst_hbm, out_hbm, acc_sp, didx, ones, zeros, vbuf, sem0, sem1,
               sem2, sem3):
    c = lax.axis_index("c")
    s = lax.axis_index("s")
    wid = s * NC + c
    rbase = s * RPT
    ssem = [sem0, sem1, sem2, sem3]
    pltpu.sync_copy(dst_hbm.at[pl.ds(wid * STEPS, STEPS)], didx)
    for j in range(W // 16):
        ones[pl.ds(j * 16, 16)] = jnp.ones((16,), _f32)
        zeros[pl.ds(j * 16, 16)] = jnp.zeros((16,), _f32)
    for k in range(RPT // W):
        pltpu.sync_copy(zeros, acc_sp.at[pl.ds(rbase + k * W, W)])
    pltpu.sync_copy(zeros.at[pl.ds(0, RPT % W)],
                    acc_sp.at[pl.ds(rbase + (RPT // W) * W, RPT % W)])
    plsc.subcore_barrier()

    def group(g, carry):
        for b in range(NBUF):
            t = g * NBUF + b

            @pl.when(g > 0)
            def _():
                pltpu.make_async_copy(ones, acc_sp.at[didx.at[t]],
                                      ssem[b]).wait()

            pltpu.async_copy(ones, acc_sp.at[didx.at[t]], ssem[b], add=True)
        return carry

    lax.fori_loop(0, GROUPS, group, 0)
    for b in range(NBUF):
        pltpu.make_async_copy(ones, acc_sp.at[didx.at[0]], ssem[b]).wait()
    plsc.subcore_barrier()
    obase = pl.multiple_of(c * N_PAD + rbase, 8)
    for k in range(RPT // W + 1):
        ln = W if k < RPT // W else RPT % W
        pltpu.sync_copy(acc_sp.at[pl.ds(rbase + k * W, ln)], vbuf.at[pl.ds(0, ln)])
        pltpu.sync_copy(vbuf.at[pl.ds(0, ln)], out_hbm.at[pl.ds(obase + k * W, ln)])


_hist = functools.partial(
    pl.kernel,
    out_type=jax.ShapeDtypeStruct((NC * N_PAD,), _f32),
    mesh=_mesh(),
    compiler_params=_SC_PARAMS,
    scratch_types=[
        pltpu.VMEM_SHARED((N_PAD,), _f32),
        pltpu.VMEM((STEPS, W), jnp.int32),
        pltpu.VMEM((W,), _f32),
        pltpu.VMEM((W,), _f32),
        pltpu.VMEM((W,), _f32),
        pltpu.SemaphoreType.DMA,
        pltpu.SemaphoreType.DMA,
        pltpu.SemaphoreType.DMA,
        pltpu.SemaphoreType.DMA,
    ],
)(_hist_body)


ANBUF = 8
LOOK = ANBUF // 2
AGROUPS = STEPS // ANBUF


def _agg_body_for(D):
    def agg(xs_hbm, src_hbm, dst_hbm, out_hbm, xs_sp, acc_sp, sidx, didx,
            *bufs):
        c = lax.axis_index("c")
        s = lax.axis_index("s")
        wid = s * NC + c
        rbase = s * RPT
        rows = list(bufs[:ANBUF])
        gsem = list(bufs[ANBUF:2 * ANBUF])
        ssem = list(bufs[2 * ANBUF:3 * ANBUF])
        pltpu.async_copy(xs_hbm.at[pl.ds(rbase, RPT)],
                         xs_sp.at[pl.ds(rbase, RPT)], gsem[0])
        pltpu.async_copy(src_hbm.at[pl.ds(wid * STEPS, STEPS)], sidx, gsem[1])
        pltpu.async_copy(dst_hbm.at[pl.ds(wid * STEPS, STEPS)], didx, gsem[2])

        def zrow(i, carry):
            for j in range(D // 16):
                rows[0][i, pl.ds(j * 16, 16)] = jnp.zeros((16,), _f32)
            return carry

        lax.fori_loop(0, W, zrow, 0)
        pltpu.make_async_copy(xs_hbm.at[pl.ds(rbase, RPT)],
                              xs_sp.at[pl.ds(rbase, RPT)], gsem[0]).wait()
        pltpu.make_async_copy(src_hbm.at[pl.ds(wid * STEPS, STEPS)], sidx,
                              gsem[1]).wait()
        pltpu.make_async_copy(dst_hbm.at[pl.ds(wid * STEPS, STEPS)], didx,
                              gsem[2]).wait()
        for k in range(RPT // W):
            pltpu.sync_copy(rows[0], acc_sp.at[pl.ds(rbase + k * W, W)])
        pltpu.sync_copy(rows[0].at[pl.ds(0, RPT % W)],
                        acc_sp.at[pl.ds(rbase + (RPT // W) * W, RPT % W)])
        plsc.subcore_barrier()

        for t0 in range(LOOK):
            pltpu.async_copy(xs_sp.at[sidx.at[t0]], rows[t0], gsem[t0])

        def group(g, carry):
            for b in range(ANBUF):
                t = g * ANBUF + b
                bp = (b + LOOK) % ANBUF
                pltpu.make_async_copy(xs_sp.at[sidx.at[t]], rows[b],
                                      gsem[b]).wait()
                pltpu.async_copy(rows[b], acc_sp.at[didx.at[t]], ssem[b],
                                 add=True)
                if b < ANBUF - LOOK:
                    @pl.when(g > 0)
                    def _():
                        pltpu.make_async_copy(rows[bp], acc_sp.at[didx.at[t]],
                                              ssem[bp]).wait()

                    pltpu.async_copy(xs_sp.at[sidx.at[t + LOOK]], rows[bp],
                                     gsem[bp])
                else:
                    @pl.when(g < AGROUPS - 1)
                    def _():
                        pltpu.make_async_copy(rows[bp], acc_sp.at[didx.at[t]],
                                              ssem[bp]).wait()
                        pltpu.async_copy(xs_sp.at[sidx.at[t + LOOK]], rows[bp],
                                         gsem[bp])
            return carry

        lax.fori_loop(0, AGROUPS, group, 0)
        for b in range(ANBUF):
            pltpu.make_async_copy(rows[b], acc_sp.at[didx.at[0]],
                                  ssem[b]).wait()
        plsc.subcore_barrier()
        pltpu.sync_copy(acc_sp.at[pl.ds(rbase, RPT)],
                        out_hbm.at[c, pl.ds(rbase, RPT)])

    return agg


def _make_agg(D):
    return functools.partial(
        pl.kernel,
        out_type=jax.ShapeDtypeStruct((NC, N_PAD, D), _f32),
        mesh=_mesh(),
        compiler_params=_SC_PARAMS,
        scratch_types=[
            pltpu.VMEM_SHARED((N_PAD, D), _f32),
            pltpu.VMEM_SHARED((N_PAD, D), _f32),
            pltpu.VMEM((STEPS, W), jnp.int32),
            pltpu.VMEM((STEPS, W), jnp.int32),
        ] + [pltpu.VMEM((W, D), _f32)] * ANBUF
          + [pltpu.SemaphoreType.DMA] * (2 * ANBUF),
    )(_agg_body_for(D))


_agg32 = _make_agg(H1)
_agg16 = _make_agg(H2)


def _edges_body(ei_ref, src_ref, dst_ref):
    pad = N + lax.rem(
        lax.broadcasted_iota(jnp.int32, (E_PAD - E,), 0), N_SPARE)
    src_ref[pl.ds(0, E)] = ei_ref[0]
    dst_ref[pl.ds(0, E)] = ei_ref[1]
    src_ref[pl.ds(E, E_PAD - E)] = pad
    dst_ref[pl.ds(E, E_PAD - E)] = pad


_edges = pl.pallas_call(
    _edges_body,
    out_shape=(jax.ShapeDtypeStruct((E_PAD,), jnp.int32),
               jax.ShapeDtypeStruct((E_PAD,), jnp.int32)),
)


def _prep_body(degp_ref, x_ref, w1_ref, dinv_ref, xs1_ref):
    deg = jnp.sum(degp_ref[...], axis=1, keepdims=True) + 1.0
    dinv = lax.rsqrt(deg)
    mask = (lax.broadcasted_iota(jnp.int32, (N_PAD, 1), 0) < N).astype(_f32)
    dinv = dinv * mask
    dinv_ref[...] = dinv
    xs1_ref[...] = jnp.dot(x_ref[...], w1_ref[...],
                           preferred_element_type=_f32) * dinv


_prep = pl.pallas_call(
    _prep_body,
    out_shape=(jax.ShapeDtypeStruct((N_PAD, 1), _f32),
               jax.ShapeDtypeStruct((N_PAD, H1), _f32)),
)


def _mid1_body(agg_ref, xs_ref, dinv_ref, b_ref, w_ref, out_ref):
    t = agg_ref[0] + agg_ref[1] + xs_ref[...]
    h = jnp.maximum(dinv_ref[...] * t + b_ref[...], 0.0)
    out_ref[...] = jnp.dot(h, w_ref[...], preferred_element_type=_f32) * dinv_ref[...]


_mid1 = pl.pallas_call(
    _mid1_body,
    out_shape=jax.ShapeDtypeStruct((N_PAD, H2), _f32),
)


def _mid2_body(agg_ref, xs_ref, dinv_ref, b_ref, out_ref):
    t = agg_ref[0] + agg_ref[1] + xs_ref[...]
    h = jnp.maximum(dinv_ref[...] * t + b_ref[...], 0.0)
    out_ref[...] = dinv_ref[...] * h


_mid2 = pl.pallas_call(
    _mid2_body,
    out_shape=jax.ShapeDtypeStruct((N_PAD, H2), _f32),
)


def _final_body(agg_ref, xs_ref, dinv_ref, w_ref, b_ref, out_ref):
    t = agg_ref[0] + agg_ref[1] + xs_ref[...]
    z = jnp.dot(dinv_ref[...] * t, w_ref[...],
                preferred_element_type=_f32) + b_ref[...]
    m = jnp.max(z, axis=1, keepdims=True)
    e = jnp.exp(z - m)
    out_ref[...] = (z - m) - jnp.log(jnp.sum(e, axis=1, keepdims=True))


_final = pl.pallas_call(
    _final_body,
    out_shape=jax.ShapeDtypeStruct((N_PAD, D_OUT), _f32),
)


def kernel(x, edge_index, W1, b1, W2, b2, W3, b3):
    src_p, dst_p = _edges(edge_index)
    src_p = src_p.reshape(NW * STEPS, W)
    dst_p = dst_p.reshape(NW * STEPS, W)
    x_p = jnp.pad(x, ((0, N_PAD - N), (0, 0)))

    degp = _hist(dst_p).reshape(NC, N_PAD)
    dinv, xs1 = _prep(degp.T, x_p, W1)
    agg1 = _agg32(xs1, src_p, dst_p)
    xs2 = _mid1(agg1, xs1, dinv, b1.reshape(1, H1), W2)
    agg2 = _agg16(xs2, src_p, dst_p)
    xs3 = _mid2(agg2, xs2, dinv, b2.reshape(1, H2))
    agg3 = _agg16(xs3, src_p, dst_p)
    out = _final(agg3, xs3, dinv, W3, b3.reshape(1, D_OUT))
    return out[:N]


# speedup vs baseline: 64.1385x; 1.0226x over previous
import functools

import jax
import jax.numpy as jnp
from jax import lax
from jax.experimental import pallas as pl
from jax.experimental.pallas import tpu as pltpu
from jax.experimental.pallas import tpu_sc as plsc

N = 10000
E = 320000
D_IN = 128
H1 = 32
H2 = 16
D_OUT = 2

NC = 2
NS = 16
NW = NC * NS

W = 128
STEPS = 80
NBUF = 4
GROUPS = STEPS // NBUF
EPW = W * STEPS
E_PAD = EPW * NW
N_PAD = 10112
RPT = N_PAD // NS
N_SPARE = N_PAD - N

_f32 = jnp.float32

_SC_PARAMS = pltpu.CompilerParams(use_tc_tiling_on_sc=False)


def _mesh():
    return plsc.VectorSubcoreMesh(core_axis_name="c", subcore_axis_name="s")


def _hist_body(dst_hbm, out_hbm, acc_sp, didx, ones, zeros, vbuf, sem0, sem1,
               sem2, sem3):
    c = lax.axis_index("c")
    s = lax.axis_index("s")
    wid = s * NC + c
    rbase = s * RPT
    ssem = [sem0, sem1, sem2, sem3]
    pltpu.sync_copy(dst_hbm.at[pl.ds(wid * STEPS, STEPS)], didx)
    for j in range(W // 16):
        ones[pl.ds(j * 16, 16)] = jnp.ones((16,), _f32)
        zeros[pl.ds(j * 16, 16)] = jnp.zeros((16,), _f32)
    for k in range(RPT // W):
        pltpu.sync_copy(zeros, acc_sp.at[pl.ds(rbase + k * W, W)])
    pltpu.sync_copy(zeros.at[pl.ds(0, RPT % W)],
                    acc_sp.at[pl.ds(rbase + (RPT // W) * W, RPT % W)])
    plsc.subcore_barrier()

    def group(g, carry):
        for b in range(NBUF):
            t = g * NBUF + b

            @pl.when(g > 0)
            def _():
                pltpu.make_async_copy(ones, acc_sp.at[didx.at[t]],
                                      ssem[b]).wait()

            pltpu.async_copy(ones, acc_sp.at[didx.at[t]], ssem[b], add=True)
        return carry

    lax.fori_loop(0, GROUPS, group, 0)
    for b in range(NBUF):
        pltpu.make_async_copy(ones, acc_sp.at[didx.at[0]], ssem[b]).wait()
    plsc.subcore_barrier()
    obase = pl.multiple_of(c * N_PAD + rbase, 8)
    for k in range(RPT // W + 1):
        ln = W if k < RPT // W else RPT % W
        pltpu.sync_copy(acc_sp.at[pl.ds(rbase + k * W, ln)], vbuf.at[pl.ds(0, ln)])
        pltpu.sync_copy(vbuf.at[pl.ds(0, ln)], out_hbm.at[pl.ds(obase + k * W, ln)])


_hist = functools.partial(
    pl.kernel,
    out_type=jax.ShapeDtypeStruct((NC * N_PAD,), _f32),
    mesh=_mesh(),
    compiler_params=_SC_PARAMS,
    scratch_types=[
        pltpu.VMEM_SHARED((N_PAD,), _f32),
        pltpu.VMEM((STEPS, W), jnp.int32),
        pltpu.VMEM((W,), _f32),
        pltpu.VMEM((W,), _f32),
        pltpu.VMEM((W,), _f32),
        pltpu.SemaphoreType.DMA,
        pltpu.SemaphoreType.DMA,
        pltpu.SemaphoreType.DMA,
        pltpu.SemaphoreType.DMA,
    ],
)(_hist_body)


ANBUF = 8
LOOK = ANBUF // 2
AGROUPS = STEPS // ANBUF


def _agg_body_for(D):
    def agg(xs_hbm, src_hbm, dst_hbm, out_hbm, xs_sp, acc_sp, sidx, didx,
            *bufs):
        c = lax.axis_index("c")
        s = lax.axis_index("s")
        wid = s * NC + c
        rbase = s * RPT
        rows = list(bufs[:ANBUF])
        gsem = list(bufs[ANBUF:2 * ANBUF])
        ssem = list(bufs[2 * ANBUF:3 * ANBUF])
        pltpu.async_copy(xs_hbm.at[pl.ds(rbase, RPT)],
                         xs_sp.at[pl.ds(rbase, RPT)], gsem[0])
        pltpu.async_copy(src_hbm.at[pl.ds(wid * STEPS, STEPS)], sidx, gsem[1])
        pltpu.async_copy(dst_hbm.at[pl.ds(wid * STEPS, STEPS)], didx, gsem[2])

        def zrow(i, carry):
            for j in range(D // 16):
                rows[0][i, pl.ds(j * 16, 16)] = jnp.zeros((16,), _f32)
            return carry

        lax.fori_loop(0, W, zrow, 0)
        pltpu.make_async_copy(xs_hbm.at[pl.ds(rbase, RPT)],
                              xs_sp.at[pl.ds(rbase, RPT)], gsem[0]).wait()
        pltpu.make_async_copy(src_hbm.at[pl.ds(wid * STEPS, STEPS)], sidx,
                              gsem[1]).wait()
        pltpu.make_async_copy(dst_hbm.at[pl.ds(wid * STEPS, STEPS)], didx,
                              gsem[2]).wait()
        for k in range(RPT // W):
            pltpu.sync_copy(rows[0], acc_sp.at[pl.ds(rbase + k * W, W)])
        pltpu.sync_copy(rows[0].at[pl.ds(0, RPT % W)],
                        acc_sp.at[pl.ds(rbase + (RPT // W) * W, RPT % W)])
        plsc.subcore_barrier()

        for t0 in range(LOOK):
            pltpu.async_copy(xs_sp.at[sidx.at[t0]], rows[t0], gsem[t0])

        def group(g, carry):
            for b in range(ANBUF):
                t = g * ANBUF + b
                bp = (b + LOOK) % ANBUF
                pltpu.make_async_copy(xs_sp.at[sidx.at[t]], rows[b],
                                      gsem[b]).wait()
                pltpu.async_copy(rows[b], acc_sp.at[didx.at[t]], ssem[b],
                                 add=True)
                if b < ANBUF - LOOK:
                    @pl.when(g > 0)
                    def _():
                        pltpu.make_async_copy(rows[bp], acc_sp.at[didx.at[t]],
                                              ssem[bp]).wait()

                    pltpu.async_copy(xs_sp.at[sidx.at[t + LOOK]], rows[bp],
                                     gsem[bp])
                else:
                    @pl.when(g < AGROUPS - 1)
                    def _():
                        pltpu.make_async_copy(rows[bp], acc_sp.at[didx.at[t]],
                                              ssem[bp]).wait()
                        pltpu.async_copy(xs_sp.at[sidx.at[t + LOOK]], rows[bp],
                                         gsem[bp])
            return carry

        lax.fori_loop(0, AGROUPS, group, 0)
        for b in range(ANBUF):
            pltpu.make_async_copy(rows[b], acc_sp.at[didx.at[0]],
                                  ssem[b]).wait()
        plsc.subcore_barrier()
        pltpu.sync_copy(acc_sp.at[pl.ds(rbase, RPT)],
                        out_hbm.at[c, pl.ds(rbase, RPT)])

    return agg


def _make_agg(D):
    return functools.partial(
        pl.kernel,
        out_type=jax.ShapeDtypeStruct((NC, N_PAD, D), _f32),
        mesh=_mesh(),
        compiler_params=_SC_PARAMS,
        scratch_types=[
            pltpu.VMEM_SHARED((N_PAD, D), _f32),
            pltpu.VMEM_SHARED((N_PAD, D), _f32),
            pltpu.VMEM((STEPS, W), jnp.int32),
            pltpu.VMEM((STEPS, W), jnp.int32),
        ] + [pltpu.VMEM((W, D), _f32)] * ANBUF
          + [pltpu.SemaphoreType.DMA] * (2 * ANBUF),
    )(_agg_body_for(D))


_agg32 = _make_agg(H1)
_agg16 = _make_agg(H2)


def _edges_body(ei_ref, src_ref, dst_ref):
    pad = N + lax.rem(
        lax.broadcasted_iota(jnp.int32, (E_PAD - E,), 0), N_SPARE)
    src_ref[pl.ds(0, E)] = ei_ref[0]
    dst_ref[pl.ds(0, E)] = ei_ref[1]
    src_ref[pl.ds(E, E_PAD - E)] = pad
    dst_ref[pl.ds(E, E_PAD - E)] = pad


_edges = pl.pallas_call(
    _edges_body,
    out_shape=(jax.ShapeDtypeStruct((E_PAD,), jnp.int32),
               jax.ShapeDtypeStruct((E_PAD,), jnp.int32)),
)


def _prep_body(degp_ref, x_ref, w1_ref, dinv_ref, xs1_ref):
    deg = jnp.sum(degp_ref[...], axis=0, keepdims=True) + 1.0
    dinv = lax.rsqrt(deg)
    mask = (lax.broadcasted_iota(jnp.int32, (1, N_PAD), 1) < N).astype(_f32)
    dinv = dinv * mask
    dinv_ref[...] = dinv
    dinv_col = jnp.transpose(dinv)
    xs1_ref[...] = jnp.dot(x_ref[...], w1_ref[...],
                           preferred_element_type=_f32) * dinv_col


_prep = pl.pallas_call(
    _prep_body,
    out_shape=(jax.ShapeDtypeStruct((1, N_PAD), _f32),
               jax.ShapeDtypeStruct((N_PAD, H1), _f32)),
)


def _mid1_body(agg_ref, xs_ref, dinv_ref, b_ref, w_ref, out_ref):
    dinv = jnp.transpose(dinv_ref[...])
    t = agg_ref[0] + agg_ref[1] + xs_ref[...]
    h = jnp.maximum(dinv * t + b_ref[...], 0.0)
    out_ref[...] = jnp.dot(h, w_ref[...], preferred_element_type=_f32) * dinv


_mid1 = pl.pallas_call(
    _mid1_body,
    out_shape=jax.ShapeDtypeStruct((N_PAD, H2), _f32),
)


def _mid2_body(agg_ref, xs_ref, dinv_ref, b_ref, out_ref):
    dinv = jnp.transpose(dinv_ref[...])
    t = agg_ref[0] + agg_ref[1] + xs_ref[...]
    h = jnp.maximum(dinv * t + b_ref[...], 0.0)
    out_ref[...] = dinv * h


_mid2 = pl.pallas_call(
    _mid2_body,
    out_shape=jax.ShapeDtypeStruct((N_PAD, H2), _f32),
)


def _final_body(agg_ref, xs_ref, dinv_ref, w_ref, b_ref, out_ref):
    dinv = jnp.transpose(dinv_ref[...])
    t = agg_ref[0] + agg_ref[1] + xs_ref[...]
    z = jnp.dot(dinv * t, w_ref[...],
                preferred_element_type=_f32) + b_ref[...]
    m = jnp.max(z, axis=1, keepdims=True)
    e = jnp.exp(z - m)
    out_ref[...] = (z - m) - jnp.log(jnp.sum(e, axis=1, keepdims=True))


_final = pl.pallas_call(
    _final_body,
    out_shape=jax.ShapeDtypeStruct((N_PAD, D_OUT), _f32),
)


def kernel(x, edge_index, W1, b1, W2, b2, W3, b3):
    src_p, dst_p = _edges(edge_index)
    src_p = src_p.reshape(NW * STEPS, W)
    dst_p = dst_p.reshape(NW * STEPS, W)
    x_p = jnp.pad(x, ((0, N_PAD - N), (0, 0)))

    degp = _hist(dst_p).reshape(NC, N_PAD)
    dinv, xs1 = _prep(degp, x_p, W1)
    agg1 = _agg32(xs1, src_p, dst_p)
    xs2 = _mid1(agg1, xs1, dinv, b1.reshape(1, H1), W2)
    agg2 = _agg16(xs2, src_p, dst_p)
    xs3 = _mid2(agg2, xs2, dinv, b2.reshape(1, H2))
    agg3 = _agg16(xs3, src_p, dst_p)
    out = _final(agg3, xs3, dinv, W3, b3.reshape(1, D_OUT))
    return out[:N]
